# baseline (speedup 1.0000x reference)
import jax
import jax.numpy as jnp
from jax import lax
from jax.experimental import pallas as pl
from jax.experimental.pallas import tpu as pltpu
from jax.experimental.pallas import tpu_sc as plsc
EMB = 32
BATCH = 16384
NC = 2
NS = 16
NW = NC * NS
BPW = BATCH // NW
CHUNK = 128
NCHUNK = BPW // CHUNK
GROUPS = BPW // 16


def _sc_body(idx_t_hbm, idx_c_hbm, tab_t_hbm, tab_c_hbm, wb_hbm, out_hbm,
             idx_t_v, idx_c_v, rows_t, rows_c, wb_v, out_v, sem):
    c = lax.axis_index("c")
    s = lax.axis_index("s")
    wid = s * NC + c
    base = wid * BPW

    pltpu.sync_copy(idx_t_hbm.at[wid], idx_t_v)
    pltpu.sync_copy(idx_c_hbm.at[wid], idx_c_v)
    pltpu.sync_copy(wb_hbm, wb_v)

    copies = []
    for j in range(NCHUNK):
        copies.append(pltpu.async_copy(
            tab_t_hbm.at[idx_t_v.at[j]],
            rows_t.at[pl.ds(j * CHUNK, CHUNK)], sem))
        copies.append(pltpu.async_copy(
            tab_c_hbm.at[idx_c_v.at[j]],
            rows_c.at[pl.ds(j * CHUNK, CHUNK)], sem))
    for cp in copies:
        cp.wait()

    wvec = wb_v[0]
    bvec = wb_v[1]
    lane = lax.iota(jnp.int32, 16)

    def group(g, carry):
        vec = jnp.zeros((16,), jnp.float32)
        for k in range(16):
            i = g * 16 + k
            p = (rows_t[i, pl.ds(0, 16)] * rows_c[i, pl.ds(0, 16)]
                 + rows_t[i, pl.ds(16, 16)] * rows_c[i, pl.ds(16, 16)])
            vec = jnp.where(lane == k, jnp.sum(p), vec)
        z = vec * wvec + bvec
        out_v[pl.ds(g * 16, 16)] = 1.0 / (1.0 + jnp.exp(-z))
        return carry

    lax.fori_loop(0, GROUPS, group, 0)

    pltpu.sync_copy(out_v, out_hbm.at[pl.ds(base, BPW)])


def _run(idx_t, idx_c, tab_t, tab_c, wb):
    mesh = plsc.VectorSubcoreMesh(core_axis_name="c", subcore_axis_name="s",
                                  num_cores=NC, num_subcores=NS)
    f = pl.kernel(
        _sc_body,
        out_type=jax.ShapeDtypeStruct((BATCH,), jnp.float32),
        mesh=mesh,
        scratch_types=[
            pltpu.VMEM((NCHUNK, CHUNK), jnp.int32),
            pltpu.VMEM((NCHUNK, CHUNK), jnp.int32),
            pltpu.VMEM((BPW, EMB), jnp.float32),
            pltpu.VMEM((BPW, EMB), jnp.float32),
            pltpu.VMEM((2, 16), jnp.float32),
            pltpu.VMEM((BPW,), jnp.float32),
            pltpu.SemaphoreType.DMA,
        ],
        compiler_params=pltpu.CompilerParams(needs_layout_passes=False,
                                             use_tc_tiling_on_sc=False),
    )
    return f(idx_t, idx_c, tab_t, tab_c, wb)


def kernel(input_target, input_context, emb_target, emb_context, W, b):
    idx_t = input_target.reshape(NW, NCHUNK, CHUNK)
    idx_c = input_context.reshape(NW, NCHUNK, CHUNK)
    wb = jnp.stack([jnp.full((16,), W[0, 0], jnp.float32),
                    jnp.full((16,), b[0], jnp.float32)])
    out = _run(idx_t, idx_c, emb_target, emb_context, wb)
    return out.reshape(BATCH, 1)
